# baseline (speedup 1.0000x reference)
import functools

import jax
import jax.numpy as jnp
from jax.experimental import pallas as pl
from jax.experimental.pallas import tpu as pltpu

K = 20
HEADS = 8
HEAD_DIM = 16
EMBED = 128
NGRAPH_PAD = 32

RCH = 256
CCH = 128
TOPW = 32
SCH = 1024
NEG = -1e30
FLOOR = -1e20


def _radius_body(lo_ref, hi_ref, posr_ref, post_ref, batr_ref, batt_ref,
                 r_ref):
    c = pl.program_id(0)
    lo = lo_ref[c]
    hi = hi_ref[c]
    px_r = posr_ref[:, 0:1]
    py_r = posr_ref[:, 1:2]
    b_r = batr_ref[...]
    row_ids = c * RCH + jax.lax.broadcasted_iota(jnp.int32, (RCH, 1), 0)
    top_d0 = jnp.full((RCH, TOPW), jnp.inf, jnp.float32)

    def col_step(jc, top_d):
        jb = jc * CCH
        px_c = post_ref[0:1, pl.ds(jb, CCH)]
        py_c = post_ref[1:2, pl.ds(jb, CCH)]
        b_c = batt_ref[0:1, pl.ds(jb, CCH)]
        col_ids = jb + jax.lax.broadcasted_iota(jnp.int32, (1, CCH), 1)
        d2 = (px_r - px_c) ** 2 + (py_r - py_c) ** 2
        bad = (b_r != b_c) | (row_ids == col_ids)
        cand = jnp.concatenate([top_d, jnp.where(bad, jnp.inf, d2)], axis=1)
        nd = []
        for _ in range(K):
            m = jnp.min(cand, axis=1, keepdims=True)
            nd.append(m)
            cand = jnp.where(cand == m, jnp.inf, cand)
        pad_d = jnp.full((RCH, TOPW - K), jnp.inf, jnp.float32)
        return jnp.concatenate(nd + [pad_d], axis=1)

    top_d = jax.lax.fori_loop(
        lo // CCH, (hi + CCH - 1) // CCH, col_step, top_d0)
    r_ref[...] = top_d[:, K - 1:K]


def _radius(pos_pad, post, batr, batt, lo, hi, Np):
    return pl.pallas_call(
        _radius_body,
        out_shape=jax.ShapeDtypeStruct((Np, 1), jnp.float32),
        grid=(Np // RCH,),
        in_specs=[
            pl.BlockSpec(memory_space=pltpu.SMEM),
            pl.BlockSpec(memory_space=pltpu.SMEM),
            pl.BlockSpec((RCH, 2), lambda c: (c, 0)),
            pl.BlockSpec((2, Np), lambda c: (0, 0)),
            pl.BlockSpec((RCH, 1), lambda c: (c, 0)),
            pl.BlockSpec((1, Np), lambda c: (0, 0)),
        ],
        out_specs=pl.BlockSpec((RCH, 1), lambda c: (c, 0)),
    )(lo, hi, pos_pad, post, batr, batt)


def _proj_body(x_ref, w_ref, b_ref, o_ref):
    o_ref[...] = jnp.dot(x_ref[...], w_ref[...],
                         preferred_element_type=jnp.float32) + b_ref[...]


def _project(x_pad, W_in, b_in, Np):
    xp = jnp.pad(x_pad, ((0, 0), (0, 5)))
    wp = jnp.pad(W_in, ((0, 5), (0, 0)))
    return pl.pallas_call(
        _proj_body,
        out_shape=jax.ShapeDtypeStruct((Np, EMBED), jnp.float32),
        grid=(Np // RCH,),
        in_specs=[pl.BlockSpec((RCH, 8), lambda i: (i, 0)),
                  pl.BlockSpec((8, EMBED), lambda i: (0, 0)),
                  pl.BlockSpec((1, EMBED), lambda i: (0, 0))],
        out_specs=pl.BlockSpec((RCH, EMBED), lambda i: (i, 0)),
    )(xp, wp, b_in.reshape(1, EMBED))


def _pass1_body(h_ref, w_ref, asrc_ref, adst_ref, sel_ref,
                xw_ref, as_ref, ad_ref):
    xw = jnp.dot(h_ref[...], w_ref[...], preferred_element_type=jnp.float32)
    xw_ref[...] = xw
    sel = sel_ref[...]
    as_ref[...] = jnp.dot(xw * asrc_ref[...], sel,
                          preferred_element_type=jnp.float32)
    ad_ref[...] = jnp.dot(xw * adst_ref[...], sel,
                          preferred_element_type=jnp.float32)


def _pass1(h, W, asrc_flat, adst_flat, Np):
    sel = (jax.lax.broadcasted_iota(jnp.int32, (EMBED, HEADS), 0) // HEAD_DIM
           == jax.lax.broadcasted_iota(jnp.int32, (EMBED, HEADS), 1)
           ).astype(jnp.float32)
    return pl.pallas_call(
        _pass1_body,
        out_shape=(jax.ShapeDtypeStruct((Np, EMBED), jnp.float32),
                   jax.ShapeDtypeStruct((Np, HEADS), jnp.float32),
                   jax.ShapeDtypeStruct((Np, HEADS), jnp.float32)),
        grid=(Np // RCH,),
        in_specs=[pl.BlockSpec((RCH, EMBED), lambda i: (i, 0)),
                  pl.BlockSpec((EMBED, EMBED), lambda i: (0, 0)),
                  pl.BlockSpec((1, EMBED), lambda i: (0, 0)),
                  pl.BlockSpec((1, EMBED), lambda i: (0, 0)),
                  pl.BlockSpec((EMBED, HEADS), lambda i: (0, 0))],
        out_specs=(pl.BlockSpec((RCH, EMBED), lambda i: (i, 0)),
                   pl.BlockSpec((RCH, HEADS), lambda i: (i, 0)),
                   pl.BlockSpec((RCH, HEADS), lambda i: (i, 0))),
    )(h, W, asrc_flat.reshape(1, EMBED), adst_flat.reshape(1, EMBED), sel)


def _pass2_body(lo_ref, hi_ref, posr_ref, post_ref, batr_ref, batt_ref,
                rr_ref, rt_ref, adt_ref, hres_ref, xw_ref, as_ref,
                prm_ref, out_ref):
    D = RCH
    nch = out_ref.shape[0] // D
    ind = (jax.lax.broadcasted_iota(jnp.int32, (HEADS, HEADS * D), 1) // D
           == jax.lax.broadcasted_iota(jnp.int32, (HEADS, HEADS * D), 0)
           ).astype(jnp.float32)
    bg = prm_ref[0:1, :]
    g = prm_ref[1:2, :]
    be = prm_ref[2:3, :]
    rm = prm_ref[3:4, :]
    rv = prm_ref[4:5, :]
    m0 = jnp.full((1, HEADS * D), FLOOR, jnp.float32)
    l0 = jnp.zeros((1, HEADS * D), jnp.float32)
    a0 = jnp.zeros((D, EMBED), jnp.float32)
    dn = (((0,), (0,)), ((), ()))

    def dst_step(c, _):
        lo = lo_ref[c]
        hi = hi_ref[c]
        db = c * D
        adt = adt_ref[:, pl.ds(db, D)]
        px_d = post_ref[0:1, pl.ds(db, D)]
        py_d = post_ref[1:2, pl.ds(db, D)]
        b_d = batt_ref[0:1, pl.ds(db, D)]
        r_d = rt_ref[0:1, pl.ds(db, D)]

        def src_step(jc, carry):
            m, l, acc = carry
            jb = jc * SCH
            px_s = posr_ref[pl.ds(jb, SCH), 0:1]
            py_s = posr_ref[pl.ds(jb, SCH), 1:2]
            b_s = batr_ref[pl.ds(jb, SCH), :]
            r_s = rr_ref[pl.ds(jb, SCH), :]
            d2 = (px_s - px_d) ** 2 + (py_s - py_d) ** 2
            mask = (b_s == b_d) & (d2 <= jnp.maximum(r_s, r_d) * (1 + 2e-6))
            madd = jnp.where(mask, 0.0, NEG)
            as_c = as_ref[pl.ds(jb, SCH), :]
            e_as = jnp.dot(as_c, ind, preferred_element_type=jnp.float32)
            nm, nl, na = [], [], []
            for h in range(HEADS):
                e = e_as[:, h * D:(h + 1) * D] + adt[h:h + 1, :]
                e = jnp.where(e > 0, e, 0.2 * e) + madd
                m_old = m[:, h * D:(h + 1) * D]
                m_new = jnp.maximum(m_old, jnp.max(e, axis=0, keepdims=True))
                p = jnp.exp(e - m_new)
                scale = jnp.exp(m_old - m_new)
                xw_h = xw_ref[pl.ds(jb, SCH), h * HEAD_DIM:(h + 1) * HEAD_DIM]
                nm.append(m_new)
                nl.append(l[:, h * D:(h + 1) * D] * scale
                          + jnp.sum(p, axis=0, keepdims=True))
                na.append(acc[:, h * HEAD_DIM:(h + 1) * HEAD_DIM]
                          * jnp.transpose(scale)
                          + jax.lax.dot_general(
                              p, xw_h, dn, preferred_element_type=jnp.float32))
            return (jnp.concatenate(nm, axis=1), jnp.concatenate(nl, axis=1),
                    jnp.concatenate(na, axis=1))

        m, l, acc = jax.lax.fori_loop(
            lo // SCH, (hi + SCH - 1) // SCH, src_step, (m0, l0, a0))
        cols = [acc[:, h * HEAD_DIM:(h + 1) * HEAD_DIM]
                / (jnp.transpose(l[:, h * D:(h + 1) * D]) + 1e-30)
                for h in range(HEADS)]
        out = jnp.concatenate(cols, axis=1)
        out = out + bg
        out = (out - rm) / jnp.sqrt(rv + 1e-5) * g + be
        out = jnp.where(out > 0, out, jnp.exp(jnp.minimum(out, 0.0)) - 1.0)
        out_ref[pl.ds(db, D), :] = out + hres_ref[pl.ds(db, D), :]
        return 0

    jax.lax.fori_loop(0, nch, dst_step, 0)


def _pass2(lo, hi, pos_pad, post, batr, batt, rr, rt, adt, h, xw, a_s,
           prm, Np):
    vspec = pl.BlockSpec(memory_space=pltpu.VMEM)
    return pl.pallas_call(
        _pass2_body,
        out_shape=jax.ShapeDtypeStruct((Np, EMBED), jnp.float32),
        in_specs=[pl.BlockSpec(memory_space=pltpu.SMEM),
                  pl.BlockSpec(memory_space=pltpu.SMEM)] + [vspec] * 11,
    )(lo, hi, pos_pad, post, batr, batt, rr, rt, adt, h, xw, a_s, prm)


def _pool_body(h_ref, bat_ref, sum_ref, cnt_ref):
    c = pl.program_id(0)

    @pl.when(c == 0)
    def _():
        sum_ref[...] = jnp.zeros_like(sum_ref)
        cnt_ref[...] = jnp.zeros_like(cnt_ref)

    onehot = (bat_ref[...] == jax.lax.broadcasted_iota(
        jnp.int32, (1, NGRAPH_PAD), 1)).astype(jnp.float32)
    dn = (((0,), (0,)), ((), ()))
    sum_ref[...] += jax.lax.dot_general(
        onehot, h_ref[...], dn, preferred_element_type=jnp.float32)
    cnt_ref[...] += jax.lax.dot_general(
        onehot, jnp.ones_like(h_ref), dn, preferred_element_type=jnp.float32)


def _pool(h, batch_r, Np):
    return pl.pallas_call(
        _pool_body,
        out_shape=(jax.ShapeDtypeStruct((NGRAPH_PAD, EMBED), jnp.float32),
                   jax.ShapeDtypeStruct((NGRAPH_PAD, EMBED), jnp.float32)),
        grid=(Np // RCH,),
        in_specs=[pl.BlockSpec((RCH, EMBED), lambda c: (c, 0)),
                  pl.BlockSpec((RCH, 1), lambda c: (c, 0))],
        out_specs=(pl.BlockSpec((NGRAPH_PAD, EMBED), lambda c: (0, 0)),
                   pl.BlockSpec((NGRAPH_PAD, EMBED), lambda c: (0, 0))),
    )(h, batch_r)


def kernel(x, pos, batch, W_in, b_in, W0, asrc0, adst0, bg0, g0, be0, rm0, rv0, W1, asrc1, adst1, bg1, g1, be1, rm1, rv1, W2, asrc2, adst2, bg2, g2, be2, rm2, rv2):
    N = x.shape[0]
    n_graphs = 25
    Np = ((N + RCH - 1) // RCH) * RCH
    padn = Np - N
    batch = batch.astype(jnp.int32)
    pos_pad = jnp.pad(pos, ((0, padn), (0, 0)))
    x_pad = jnp.pad(x, ((0, padn), (0, 0)))
    batch_pad = jnp.pad(batch, (0, padn), constant_values=127)

    gids = jnp.arange(n_graphs, dtype=jnp.int32)
    starts = jnp.searchsorted(batch, gids, side='left').astype(jnp.int32)
    ends = jnp.searchsorted(batch, gids, side='right').astype(jnp.int32)
    nch = Np // RCH
    r0 = jnp.arange(nch, dtype=jnp.int32) * RCH
    r1 = jnp.minimum(r0 + RCH, N) - 1
    live = r0 < N
    b0 = batch[jnp.clip(r0, 0, N - 1)]
    b1 = batch[jnp.clip(r1, 0, N - 1)]
    lo = jnp.where(live, starts[b0], 0)
    hi = jnp.where(live, ends[b1], 0)

    post = jnp.transpose(pos_pad).reshape(2, Np)
    batr = batch_pad.reshape(Np, 1)
    batt = batch_pad.reshape(1, Np)
    rr = _radius(pos_pad, post, batr, batt, lo, hi, Np)
    rt = rr.reshape(1, Np)

    h = _project(x_pad, W_in, b_in, Np)
    batch_r = batr

    layers = [(W0, asrc0, adst0, bg0, g0, be0, rm0, rv0),
              (W1, asrc1, adst1, bg1, g1, be1, rm1, rv1),
              (W2, asrc2, adst2, bg2, g2, be2, rm2, rv2)]
    for (W, asrc, adst, bg, g, be, rm, rv) in layers:
        xw, a_s, a_d = _pass1(h, W, asrc.reshape(-1), adst.reshape(-1), Np)
        adt = jnp.transpose(a_d).reshape(HEADS, Np)
        prm = jnp.stack([bg, g, be, rm, rv, bg, bg, bg], axis=0)
        h = _pass2(lo, hi, pos_pad, post, batr, batt, rr, rt, adt, h, xw,
                   a_s, prm, Np)

    sums, cnts = _pool(h, batch_r, Np)
    node_emb = h[:N]
    graph_emb = sums[:n_graphs] / jnp.maximum(cnts[:n_graphs], 1.0)
    return (node_emb, graph_emb)


# speedup vs baseline: 1.0453x; 1.0453x over previous
import functools

import jax
import jax.numpy as jnp
from jax.experimental import pallas as pl
from jax.experimental.pallas import tpu as pltpu

K = 20
HEADS = 8
HEAD_DIM = 16
EMBED = 128
NGRAPH_PAD = 32

RCH = 256
CCH = 128
TOPW = 32
SCH = 512
NEG = -1e30
FLOOR = -1e20


def _radius_body(lo_ref, hi_ref, posr_ref, post_ref, batr_ref, batt_ref,
                 r_ref):
    c = pl.program_id(0)
    lo = lo_ref[c]
    hi = hi_ref[c]
    px_r = posr_ref[:, 0:1]
    py_r = posr_ref[:, 1:2]
    b_r = batr_ref[...]
    row_ids = c * RCH + jax.lax.broadcasted_iota(jnp.int32, (RCH, 1), 0)
    top_d0 = jnp.full((RCH, TOPW), jnp.inf, jnp.float32)

    def col_step(jc, top_d):
        jb = jc * CCH
        px_c = post_ref[0:1, pl.ds(jb, CCH)]
        py_c = post_ref[1:2, pl.ds(jb, CCH)]
        b_c = batt_ref[0:1, pl.ds(jb, CCH)]
        col_ids = jb + jax.lax.broadcasted_iota(jnp.int32, (1, CCH), 1)
        d2 = (px_r - px_c) ** 2 + (py_r - py_c) ** 2
        bad = (b_r != b_c) | (row_ids == col_ids)
        cand = jnp.concatenate([top_d, jnp.where(bad, jnp.inf, d2)], axis=1)
        nd = []
        for _ in range(K):
            m = jnp.min(cand, axis=1, keepdims=True)
            nd.append(m)
            cand = jnp.where(cand == m, jnp.inf, cand)
        pad_d = jnp.full((RCH, TOPW - K), jnp.inf, jnp.float32)
        return jnp.concatenate(nd + [pad_d], axis=1)

    top_d = jax.lax.fori_loop(
        lo // CCH, (hi + CCH - 1) // CCH, col_step, top_d0)
    r_ref[...] = top_d[:, K - 1:K]


def _radius(pos_pad, post, batr, batt, lo, hi, Np):
    return pl.pallas_call(
        _radius_body,
        out_shape=jax.ShapeDtypeStruct((Np, 1), jnp.float32),
        grid=(Np // RCH,),
        in_specs=[
            pl.BlockSpec(memory_space=pltpu.SMEM),
            pl.BlockSpec(memory_space=pltpu.SMEM),
            pl.BlockSpec((RCH, 2), lambda c: (c, 0)),
            pl.BlockSpec((2, Np), lambda c: (0, 0)),
            pl.BlockSpec((RCH, 1), lambda c: (c, 0)),
            pl.BlockSpec((1, Np), lambda c: (0, 0)),
        ],
        out_specs=pl.BlockSpec((RCH, 1), lambda c: (c, 0)),
    )(lo, hi, pos_pad, post, batr, batt)


def _proj_body(x_ref, w_ref, b_ref, o_ref):
    o_ref[...] = jnp.dot(x_ref[...], w_ref[...],
                         preferred_element_type=jnp.float32) + b_ref[...]


def _project(x_pad, W_in, b_in, Np):
    xp = jnp.pad(x_pad, ((0, 0), (0, 5)))
    wp = jnp.pad(W_in, ((0, 5), (0, 0)))
    return pl.pallas_call(
        _proj_body,
        out_shape=jax.ShapeDtypeStruct((Np, EMBED), jnp.float32),
        grid=(Np // RCH,),
        in_specs=[pl.BlockSpec((RCH, 8), lambda i: (i, 0)),
                  pl.BlockSpec((8, EMBED), lambda i: (0, 0)),
                  pl.BlockSpec((1, EMBED), lambda i: (0, 0))],
        out_specs=pl.BlockSpec((RCH, EMBED), lambda i: (i, 0)),
    )(xp, wp, b_in.reshape(1, EMBED))


def _pass1_body(h_ref, w_ref, asrc_ref, adst_ref, sel_ref,
                xw_ref, as_ref, ad_ref):
    xw = jnp.dot(h_ref[...], w_ref[...], preferred_element_type=jnp.float32)
    xw_ref[...] = xw
    sel = sel_ref[...]
    as_ref[...] = jnp.dot(xw * asrc_ref[...], sel,
                          preferred_element_type=jnp.float32)
    ad_ref[...] = jnp.dot(xw * adst_ref[...], sel,
                          preferred_element_type=jnp.float32)


def _pass1(h, W, asrc_flat, adst_flat, Np):
    sel = (jax.lax.broadcasted_iota(jnp.int32, (EMBED, HEADS), 0) // HEAD_DIM
           == jax.lax.broadcasted_iota(jnp.int32, (EMBED, HEADS), 1)
           ).astype(jnp.float32)
    return pl.pallas_call(
        _pass1_body,
        out_shape=(jax.ShapeDtypeStruct((Np, EMBED), jnp.float32),
                   jax.ShapeDtypeStruct((Np, HEADS), jnp.float32),
                   jax.ShapeDtypeStruct((Np, HEADS), jnp.float32)),
        grid=(Np // RCH,),
        in_specs=[pl.BlockSpec((RCH, EMBED), lambda i: (i, 0)),
                  pl.BlockSpec((EMBED, EMBED), lambda i: (0, 0)),
                  pl.BlockSpec((1, EMBED), lambda i: (0, 0)),
                  pl.BlockSpec((1, EMBED), lambda i: (0, 0)),
                  pl.BlockSpec((EMBED, HEADS), lambda i: (0, 0))],
        out_specs=(pl.BlockSpec((RCH, EMBED), lambda i: (i, 0)),
                   pl.BlockSpec((RCH, HEADS), lambda i: (i, 0)),
                   pl.BlockSpec((RCH, HEADS), lambda i: (i, 0))),
    )(h, W, asrc_flat.reshape(1, EMBED), adst_flat.reshape(1, EMBED), sel)


def _pass2_body(lo_ref, hi_ref, posr_ref, post_ref, batr_ref, batt_ref,
                rr_ref, rt_ref, adt_ref, hres_ref, xw_ref, as_ref,
                prm_ref, out_ref):
    D = RCH
    nch = out_ref.shape[0] // D
    ind = (jax.lax.broadcasted_iota(jnp.int32, (HEADS, HEADS * D), 1) // D
           == jax.lax.broadcasted_iota(jnp.int32, (HEADS, HEADS * D), 0)
           ).astype(jnp.float32)
    bg = prm_ref[0:1, :]
    g = prm_ref[1:2, :]
    be = prm_ref[2:3, :]
    rm = prm_ref[3:4, :]
    rv = prm_ref[4:5, :]
    m0 = jnp.full((1, HEADS * D), FLOOR, jnp.float32)
    l0 = jnp.zeros((1, HEADS * D), jnp.float32)
    a0 = jnp.zeros((D, EMBED), jnp.float32)
    dn = (((0,), (0,)), ((), ()))

    def dst_step(c, _):
        lo = lo_ref[c]
        hi = hi_ref[c]
        db = c * D
        adt = adt_ref[:, pl.ds(db, D)]
        px_d = post_ref[0:1, pl.ds(db, D)]
        py_d = post_ref[1:2, pl.ds(db, D)]
        b_d = batt_ref[0:1, pl.ds(db, D)]
        r_d = rt_ref[0:1, pl.ds(db, D)]

        def src_step(jc, carry):
            m, l, acc = carry
            jb = jc * SCH
            px_s = posr_ref[pl.ds(jb, SCH), 0:1]
            py_s = posr_ref[pl.ds(jb, SCH), 1:2]
            b_s = batr_ref[pl.ds(jb, SCH), :]
            r_s = rr_ref[pl.ds(jb, SCH), :]
            d2 = (px_s - px_d) ** 2 + (py_s - py_d) ** 2
            mask = (b_s == b_d) & (d2 <= jnp.maximum(r_s, r_d) * (1 + 2e-6))
            madd = jnp.where(mask, 0.0, NEG)
            as_c = as_ref[pl.ds(jb, SCH), :]
            e_as = jnp.dot(as_c, ind, preferred_element_type=jnp.float32)
            nm, nl, na = [], [], []
            for h in range(HEADS):
                e = e_as[:, h * D:(h + 1) * D] + adt[h:h + 1, :]
                e = jnp.where(e > 0, e, 0.2 * e) + madd
                m_old = m[:, h * D:(h + 1) * D]
                m_new = jnp.maximum(m_old, jnp.max(e, axis=0, keepdims=True))
                p = jnp.exp(e - m_new)
                scale = jnp.exp(m_old - m_new)
                xw_h = xw_ref[pl.ds(jb, SCH), h * HEAD_DIM:(h + 1) * HEAD_DIM]
                nm.append(m_new)
                nl.append(l[:, h * D:(h + 1) * D] * scale
                          + jnp.sum(p, axis=0, keepdims=True))
                na.append(acc[:, h * HEAD_DIM:(h + 1) * HEAD_DIM]
                          * jnp.transpose(scale)
                          + jax.lax.dot_general(
                              p, xw_h, dn, preferred_element_type=jnp.float32))
            return (jnp.concatenate(nm, axis=1), jnp.concatenate(nl, axis=1),
                    jnp.concatenate(na, axis=1))

        m, l, acc = jax.lax.fori_loop(
            lo // SCH, (hi + SCH - 1) // SCH, src_step, (m0, l0, a0))
        cols = [acc[:, h * HEAD_DIM:(h + 1) * HEAD_DIM]
                / (jnp.transpose(l[:, h * D:(h + 1) * D]) + 1e-30)
                for h in range(HEADS)]
        out = jnp.concatenate(cols, axis=1)
        out = out + bg
        out = (out - rm) / jnp.sqrt(rv + 1e-5) * g + be
        out = jnp.where(out > 0, out, jnp.exp(jnp.minimum(out, 0.0)) - 1.0)
        out_ref[pl.ds(db, D), :] = out + hres_ref[pl.ds(db, D), :]
        return 0

    jax.lax.fori_loop(0, nch, dst_step, 0)


def _pass2(lo, hi, pos_pad, post, batr, batt, rr, rt, adt, h, xw, a_s,
           prm, Np):
    vspec = pl.BlockSpec(memory_space=pltpu.VMEM)
    return pl.pallas_call(
        _pass2_body,
        out_shape=jax.ShapeDtypeStruct((Np, EMBED), jnp.float32),
        in_specs=[pl.BlockSpec(memory_space=pltpu.SMEM),
                  pl.BlockSpec(memory_space=pltpu.SMEM)] + [vspec] * 11,
    )(lo, hi, pos_pad, post, batr, batt, rr, rt, adt, h, xw, a_s, prm)


def _pool_body(h_ref, bat_ref, sum_ref, cnt_ref):
    c = pl.program_id(0)

    @pl.when(c == 0)
    def _():
        sum_ref[...] = jnp.zeros_like(sum_ref)
        cnt_ref[...] = jnp.zeros_like(cnt_ref)

    onehot = (bat_ref[...] == jax.lax.broadcasted_iota(
        jnp.int32, (1, NGRAPH_PAD), 1)).astype(jnp.float32)
    dn = (((0,), (0,)), ((), ()))
    sum_ref[...] += jax.lax.dot_general(
        onehot, h_ref[...], dn, preferred_element_type=jnp.float32)
    cnt_ref[...] += jax.lax.dot_general(
        onehot, jnp.ones_like(h_ref), dn, preferred_element_type=jnp.float32)


def _pool(h, batch_r, Np):
    return pl.pallas_call(
        _pool_body,
        out_shape=(jax.ShapeDtypeStruct((NGRAPH_PAD, EMBED), jnp.float32),
                   jax.ShapeDtypeStruct((NGRAPH_PAD, EMBED), jnp.float32)),
        grid=(Np // RCH,),
        in_specs=[pl.BlockSpec((RCH, EMBED), lambda c: (c, 0)),
                  pl.BlockSpec((RCH, 1), lambda c: (c, 0))],
        out_specs=(pl.BlockSpec((NGRAPH_PAD, EMBED), lambda c: (0, 0)),
                   pl.BlockSpec((NGRAPH_PAD, EMBED), lambda c: (0, 0))),
    )(h, batch_r)


def kernel(x, pos, batch, W_in, b_in, W0, asrc0, adst0, bg0, g0, be0, rm0, rv0, W1, asrc1, adst1, bg1, g1, be1, rm1, rv1, W2, asrc2, adst2, bg2, g2, be2, rm2, rv2):
    N = x.shape[0]
    n_graphs = 25
    Np = ((N + SCH - 1) // SCH) * SCH
    padn = Np - N
    batch = batch.astype(jnp.int32)
    pos_pad = jnp.pad(pos, ((0, padn), (0, 0)))
    x_pad = jnp.pad(x, ((0, padn), (0, 0)))
    batch_pad = jnp.pad(batch, (0, padn), constant_values=127)

    gids = jnp.arange(n_graphs, dtype=jnp.int32)
    starts = jnp.searchsorted(batch, gids, side='left').astype(jnp.int32)
    ends = jnp.searchsorted(batch, gids, side='right').astype(jnp.int32)
    nch = Np // RCH
    r0 = jnp.arange(nch, dtype=jnp.int32) * RCH
    r1 = jnp.minimum(r0 + RCH, N) - 1
    live = r0 < N
    b0 = batch[jnp.clip(r0, 0, N - 1)]
    b1 = batch[jnp.clip(r1, 0, N - 1)]
    lo = jnp.where(live, starts[b0], 0)
    hi = jnp.where(live, ends[b1], 0)

    post = jnp.transpose(pos_pad).reshape(2, Np)
    batr = batch_pad.reshape(Np, 1)
    batt = batch_pad.reshape(1, Np)
    rr = _radius(pos_pad, post, batr, batt, lo, hi, Np)
    rt = rr.reshape(1, Np)

    h = _project(x_pad, W_in, b_in, Np)
    batch_r = batr

    layers = [(W0, asrc0, adst0, bg0, g0, be0, rm0, rv0),
              (W1, asrc1, adst1, bg1, g1, be1, rm1, rv1),
              (W2, asrc2, adst2, bg2, g2, be2, rm2, rv2)]
    for (W, asrc, adst, bg, g, be, rm, rv) in layers:
        xw, a_s, a_d = _pass1(h, W, asrc.reshape(-1), adst.reshape(-1), Np)
        adt = jnp.transpose(a_d).reshape(HEADS, Np)
        prm = jnp.stack([bg, g, be, rm, rv, bg, bg, bg], axis=0)
        h = _pass2(lo, hi, pos_pad, post, batr, batt, rr, rt, adt, h, xw,
                   a_s, prm, Np)

    sums, cnts = _pool(h, batch_r, Np)
    node_emb = h[:N]
    graph_emb = sums[:n_graphs] / jnp.maximum(cnts[:n_graphs], 1.0)
    return (node_emb, graph_emb)


# speedup vs baseline: 1.0467x; 1.0013x over previous
import jax
import jax.numpy as jnp
from jax.experimental import pallas as pl
from jax.experimental.pallas import tpu as pltpu

K = 20
HEADS = 8
HEAD_DIM = 16
EMBED = 128
NGRAPH_PAD = 32

RCH = 256
CCH = 128
TOPW = 32
SCH = 512
NEG = -1e30
FLOOR = -1e20


def _radius_body(lo_ref, hi_ref, posr_ref, post_ref, batr_ref, batt_ref,
                 r_ref):
    c = pl.program_id(0)
    lo = lo_ref[c]
    hi = hi_ref[c]
    px_r = posr_ref[:, 0:1]
    py_r = posr_ref[:, 1:2]
    b_r = batr_ref[...]
    row_ids = c * RCH + jax.lax.broadcasted_iota(jnp.int32, (RCH, 1), 0)
    top_d0 = jnp.full((RCH, TOPW), jnp.inf, jnp.float32)

    def col_step(jc, top_d):
        jb = jc * CCH
        px_c = post_ref[0:1, pl.ds(jb, CCH)]
        py_c = post_ref[1:2, pl.ds(jb, CCH)]
        b_c = batt_ref[0:1, pl.ds(jb, CCH)]
        col_ids = jb + jax.lax.broadcasted_iota(jnp.int32, (1, CCH), 1)
        d2 = (px_r - px_c) ** 2 + (py_r - py_c) ** 2
        bad = (b_r != b_c) | (row_ids == col_ids)
        cand = jnp.concatenate([top_d, jnp.where(bad, jnp.inf, d2)], axis=1)
        nd = []
        for _ in range(K):
            m = jnp.min(cand, axis=1, keepdims=True)
            nd.append(m)
            cand = jnp.where(cand == m, jnp.inf, cand)
        pad_d = jnp.full((RCH, TOPW - K), jnp.inf, jnp.float32)
        return jnp.concatenate(nd + [pad_d], axis=1)

    top_d = jax.lax.fori_loop(
        lo // CCH, (hi + CCH - 1) // CCH, col_step, top_d0)
    r_ref[...] = top_d[:, K - 1:K]


def _radius(pos_pad, post, batr, batt, lo, hi, Np):
    return pl.pallas_call(
        _radius_body,
        out_shape=jax.ShapeDtypeStruct((Np, 1), jnp.float32),
        grid=(Np // RCH,),
        in_specs=[
            pl.BlockSpec(memory_space=pltpu.SMEM),
            pl.BlockSpec(memory_space=pltpu.SMEM),
            pl.BlockSpec((RCH, 2), lambda c: (c, 0)),
            pl.BlockSpec((2, Np), lambda c: (0, 0)),
            pl.BlockSpec((RCH, 1), lambda c: (c, 0)),
            pl.BlockSpec((1, Np), lambda c: (0, 0)),
        ],
        out_specs=pl.BlockSpec((RCH, 1), lambda c: (c, 0)),
    )(lo, hi, pos_pad, post, batr, batt)


def _proj_body(x_ref, w_ref, b_ref, o_ref):
    o_ref[...] = jnp.dot(x_ref[...], w_ref[...],
                         preferred_element_type=jnp.float32) + b_ref[...]


def _project(x_pad, W_in, b_in, Np):
    xp = jnp.pad(x_pad, ((0, 0), (0, 5)))
    wp = jnp.pad(W_in, ((0, 5), (0, 0)))
    return pl.pallas_call(
        _proj_body,
        out_shape=jax.ShapeDtypeStruct((Np, EMBED), jnp.float32),
        grid=(Np // RCH,),
        in_specs=[pl.BlockSpec((RCH, 8), lambda i: (i, 0)),
                  pl.BlockSpec((8, EMBED), lambda i: (0, 0)),
                  pl.BlockSpec((1, EMBED), lambda i: (0, 0))],
        out_specs=pl.BlockSpec((RCH, EMBED), lambda i: (i, 0)),
    )(xp, wp, b_in.reshape(1, EMBED))


def _pass1_body(h_ref, w_ref, asrc_ref, adst_ref, sel_ref,
                xw_ref, as_ref, ad_ref):
    xw = jnp.dot(h_ref[...], w_ref[...], preferred_element_type=jnp.float32)
    xw_ref[...] = xw
    sel = sel_ref[...]
    as_ref[...] = jnp.dot(xw * asrc_ref[...], sel,
                          preferred_element_type=jnp.float32)
    ad_ref[...] = jnp.dot(xw * adst_ref[...], sel,
                          preferred_element_type=jnp.float32)


def _pass1(h, W, asrc_flat, adst_flat, Np):
    sel = (jax.lax.broadcasted_iota(jnp.int32, (EMBED, HEADS), 0) // HEAD_DIM
           == jax.lax.broadcasted_iota(jnp.int32, (EMBED, HEADS), 1)
           ).astype(jnp.float32)
    return pl.pallas_call(
        _pass1_body,
        out_shape=(jax.ShapeDtypeStruct((Np, EMBED), jnp.float32),
                   jax.ShapeDtypeStruct((Np, HEADS), jnp.float32),
                   jax.ShapeDtypeStruct((Np, HEADS), jnp.float32)),
        grid=(Np // RCH,),
        in_specs=[pl.BlockSpec((RCH, EMBED), lambda i: (i, 0)),
                  pl.BlockSpec((EMBED, EMBED), lambda i: (0, 0)),
                  pl.BlockSpec((1, EMBED), lambda i: (0, 0)),
                  pl.BlockSpec((1, EMBED), lambda i: (0, 0)),
                  pl.BlockSpec((EMBED, HEADS), lambda i: (0, 0))],
        out_specs=(pl.BlockSpec((RCH, EMBED), lambda i: (i, 0)),
                   pl.BlockSpec((RCH, HEADS), lambda i: (i, 0)),
                   pl.BlockSpec((RCH, HEADS), lambda i: (i, 0))),
    )(h, W, asrc_flat.reshape(1, EMBED), adst_flat.reshape(1, EMBED), sel)


def _pass2_body(lo_ref, hi_ref, posr_ref, post_ref, batr_ref, batt_ref,
                rr_ref, rt_ref, adt_ref, hres_ref, xw_ref, as_ref,
                prm_ref, out_ref):
    D = RCH
    nch = out_ref.shape[0] // D
    ind = (jax.lax.broadcasted_iota(jnp.int32, (HEADS, HEADS * D), 1) // D
           == jax.lax.broadcasted_iota(jnp.int32, (HEADS, HEADS * D), 0)
           ).astype(jnp.float32)
    bg = prm_ref[0:1, :]
    g = prm_ref[1:2, :]
    be = prm_ref[2:3, :]
    rm = prm_ref[3:4, :]
    rv = prm_ref[4:5, :]
    m0 = jnp.full((1, HEADS * D), FLOOR, jnp.float32)
    l0 = jnp.zeros((1, HEADS * D), jnp.float32)
    a0 = jnp.zeros((D, EMBED), jnp.float32)
    dn = (((0,), (0,)), ((), ()))

    def dst_step(c, _):
        lo = lo_ref[c]
        hi = hi_ref[c]
        db = c * D
        adt = adt_ref[:, pl.ds(db, D)]
        px_d = post_ref[0:1, pl.ds(db, D)]
        py_d = post_ref[1:2, pl.ds(db, D)]
        b_d = batt_ref[0:1, pl.ds(db, D)]
        r_d = rt_ref[0:1, pl.ds(db, D)]

        def src_step(jc, carry):
            m, l, acc = carry
            jb = jc * SCH
            px_s = posr_ref[pl.ds(jb, SCH), 0:1]
            py_s = posr_ref[pl.ds(jb, SCH), 1:2]
            b_s = batr_ref[pl.ds(jb, SCH), :]
            r_s = rr_ref[pl.ds(jb, SCH), :]
            d2 = (px_s - px_d) ** 2 + (py_s - py_d) ** 2
            mask = (b_s == b_d) & (d2 <= jnp.maximum(r_s, r_d) * (1 + 2e-6))
            madd = jnp.where(mask, 0.0, NEG)
            as_c = as_ref[pl.ds(jb, SCH), :]
            e_as = jnp.dot(as_c, ind, preferred_element_type=jnp.float32)
            nm, nl, na = [], [], []
            for h in range(HEADS):
                e = e_as[:, h * D:(h + 1) * D] + adt[h:h + 1, :]
                e = jnp.where(e > 0, e, 0.2 * e) + madd
                m_old = m[:, h * D:(h + 1) * D]
                m_new = jnp.maximum(m_old, jnp.max(e, axis=0, keepdims=True))
                p = jnp.exp(e - m_new)
                scale = jnp.exp(m_old - m_new)
                xw_h = xw_ref[pl.ds(jb, SCH), h * HEAD_DIM:(h + 1) * HEAD_DIM]
                nm.append(m_new)
                nl.append(l[:, h * D:(h + 1) * D] * scale
                          + jnp.sum(p, axis=0, keepdims=True))
                na.append(acc[:, h * HEAD_DIM:(h + 1) * HEAD_DIM]
                          * jnp.transpose(scale)
                          + jax.lax.dot_general(
                              p, xw_h, dn, preferred_element_type=jnp.float32))
            return (jnp.concatenate(nm, axis=1), jnp.concatenate(nl, axis=1),
                    jnp.concatenate(na, axis=1))

        m, l, acc = jax.lax.fori_loop(
            lo // SCH, (hi + SCH - 1) // SCH, src_step, (m0, l0, a0))
        cols = [acc[:, h * HEAD_DIM:(h + 1) * HEAD_DIM]
                / (jnp.transpose(l[:, h * D:(h + 1) * D]) + 1e-30)
                for h in range(HEADS)]
        out = jnp.concatenate(cols, axis=1)
        out = out + bg
        out = (out - rm) / jnp.sqrt(rv + 1e-5) * g + be
        out = jnp.where(out > 0, out, jnp.exp(jnp.minimum(out, 0.0)) - 1.0)
        out_ref[pl.ds(db, D), :] = out + hres_ref[pl.ds(db, D), :]
        return 0

    jax.lax.fori_loop(0, nch, dst_step, 0)


def _pass2(lo, hi, pos_pad, post, batr, batt, rr, rt, adt, h, xw, a_s,
           prm, Np):
    vspec = pl.BlockSpec(memory_space=pltpu.VMEM)
    return pl.pallas_call(
        _pass2_body,
        out_shape=jax.ShapeDtypeStruct((Np, EMBED), jnp.float32),
        in_specs=[pl.BlockSpec(memory_space=pltpu.SMEM),
                  pl.BlockSpec(memory_space=pltpu.SMEM)] + [vspec] * 11,
    )(lo, hi, pos_pad, post, batr, batt, rr, rt, adt, h, xw, a_s, prm)


def _pool_body(h_ref, bat_ref, sum_ref, cnt_ref):
    c = pl.program_id(0)

    @pl.when(c == 0)
    def _():
        sum_ref[...] = jnp.zeros_like(sum_ref)
        cnt_ref[...] = jnp.zeros_like(cnt_ref)

    onehot = (bat_ref[...] == jax.lax.broadcasted_iota(
        jnp.int32, (1, NGRAPH_PAD), 1)).astype(jnp.float32)
    dn = (((0,), (0,)), ((), ()))
    sum_ref[...] += jax.lax.dot_general(
        onehot, h_ref[...], dn, preferred_element_type=jnp.float32)
    cnt_ref[...] += jax.lax.dot_general(
        onehot, jnp.ones_like(h_ref), dn, preferred_element_type=jnp.float32)


def _pool(h, batch_r, Np):
    return pl.pallas_call(
        _pool_body,
        out_shape=(jax.ShapeDtypeStruct((NGRAPH_PAD, EMBED), jnp.float32),
                   jax.ShapeDtypeStruct((NGRAPH_PAD, EMBED), jnp.float32)),
        grid=(Np // RCH,),
        in_specs=[pl.BlockSpec((RCH, EMBED), lambda c: (c, 0)),
                  pl.BlockSpec((RCH, 1), lambda c: (c, 0))],
        out_specs=(pl.BlockSpec((NGRAPH_PAD, EMBED), lambda c: (0, 0)),
                   pl.BlockSpec((NGRAPH_PAD, EMBED), lambda c: (0, 0))),
    )(h, batch_r)


def kernel(x, pos, batch, W_in, b_in, W0, asrc0, adst0, bg0, g0, be0, rm0, rv0, W1, asrc1, adst1, bg1, g1, be1, rm1, rv1, W2, asrc2, adst2, bg2, g2, be2, rm2, rv2):
    N = x.shape[0]
    n_graphs = 25
    Np = ((N + SCH - 1) // SCH) * SCH
    padn = Np - N
    batch = batch.astype(jnp.int32)
    pos_pad = jnp.pad(pos, ((0, padn), (0, 0)))
    x_pad = jnp.pad(x, ((0, padn), (0, 0)))
    batch_pad = jnp.pad(batch, (0, padn), constant_values=127)

    gids = jnp.arange(n_graphs, dtype=jnp.int32)
    starts = jnp.searchsorted(batch, gids, side='left').astype(jnp.int32)
    ends = jnp.searchsorted(batch, gids, side='right').astype(jnp.int32)
    nch = Np // RCH
    r0 = jnp.arange(nch, dtype=jnp.int32) * RCH
    r1 = jnp.minimum(r0 + RCH, N) - 1
    live = r0 < N
    b0 = batch[jnp.clip(r0, 0, N - 1)]
    b1 = batch[jnp.clip(r1, 0, N - 1)]
    lo = jnp.where(live, starts[b0], 0)
    hi = jnp.where(live, ends[b1], 0)

    post = jnp.transpose(pos_pad).reshape(2, Np)
    batr = batch_pad.reshape(Np, 1)
    batt = batch_pad.reshape(1, Np)
    rr = _radius(pos_pad, post, batr, batt, lo, hi, Np)
    rt = rr.reshape(1, Np)

    h = _project(x_pad, W_in, b_in, Np)
    batch_r = batr

    layers = [(W0, asrc0, adst0, bg0, g0, be0, rm0, rv0),
              (W1, asrc1, adst1, bg1, g1, be1, rm1, rv1),
              (W2, asrc2, adst2, bg2, g2, be2, rm2, rv2)]
    for (W, asrc, adst, bg, g, be, rm, rv) in layers:
        xw, a_s, a_d = _pass1(h, W, asrc.reshape(-1), adst.reshape(-1), Np)
        adt = jnp.transpose(a_d).reshape(HEADS, Np)
        prm = jnp.stack([bg, g, be, rm, rv, bg, bg, bg], axis=0)
        h = _pass2(lo, hi, pos_pad, post, batr, batt, rr, rt, adt, h, xw,
                   a_s, prm, Np)

    sums, cnts = _pool(h, batch_r, Np)
    node_emb = h[:N]
    graph_emb = sums[:n_graphs] / jnp.maximum(cnts[:n_graphs], 1.0)
    return (node_emb, graph_emb)
